# Initial kernel scaffold; baseline (speedup 1.0000x reference)
import jax
import jax.numpy as jnp
from jax.experimental import pallas as pl


def kernel(path, table):
    raise NotImplementedError("write your pallas kernel here")


# speedup vs baseline: 1.1027x; 1.1027x over previous
import functools

import jax
import jax.numpy as jnp
from jax import lax
from jax.experimental import pallas as pl
from jax.experimental.pallas import tpu as pltpu
from jax.experimental.pallas import tpu_sc as plsc

VOCAB = 361
DIM = 512
NC, NS = 2, 16
NW = NC * NS
B_TOTAL = 4 * 1024 * 50
BPW = B_TOTAL // NW
CH = 80
NCHUNK = BPW // CH

_mesh = plsc.VectorSubcoreMesh(core_axis_name="c", subcore_axis_name="s")


@functools.partial(
    pl.kernel,
    mesh=_mesh,
    out_type=jax.ShapeDtypeStruct((B_TOTAL, DIM), jnp.float32),
    scratch_types=[
        pltpu.VMEM((BPW,), jnp.int32),
        pltpu.VMEM((CH, DIM), jnp.float32),
        pltpu.SemaphoreType.DMA,
    ],
)
def _gather(idx_hbm, table_hbm, out_hbm, idx_v, rows_v, sem):
    wid = lax.axis_index("s") * NC + lax.axis_index("c")
    base = wid * BPW
    pltpu.sync_copy(idx_hbm.at[pl.ds(base, BPW)], idx_v)

    def body(c, carry):
        off = c * CH
        pltpu.async_copy(
            table_hbm.at[idx_v.at[pl.ds(off, CH)]], rows_v, sem
        ).wait()
        pltpu.sync_copy(rows_v, out_hbm.at[pl.ds(base + off, CH)])
        return carry

    lax.fori_loop(0, NCHUNK, body, 0)


def kernel(path, table):
    flat = path.reshape(-1).astype(jnp.int32)
    out = _gather(flat, table)
    out4 = out.reshape(path.shape[0], path.shape[1], path.shape[2], DIM)
    return tuple(out4[i] for i in range(path.shape[0]))


# speedup vs baseline: 1.1055x; 1.0025x over previous
import functools

import jax
import jax.numpy as jnp
from jax import lax
from jax.experimental import pallas as pl
from jax.experimental.pallas import tpu as pltpu
from jax.experimental.pallas import tpu_sc as plsc

VOCAB = 361
DIM = 512
NC, NS = 2, 16
NW = NC * NS
B_TOTAL = 4 * 1024 * 50
BPW = B_TOTAL // NW
CH = 80
NCHUNK = BPW // CH

_mesh = plsc.VectorSubcoreMesh(core_axis_name="c", subcore_axis_name="s")


@functools.partial(
    pl.kernel,
    mesh=_mesh,
    out_type=jax.ShapeDtypeStruct((B_TOTAL, DIM), jnp.float32),
    scratch_types=[
        pltpu.VMEM((BPW,), jnp.int32),
        pltpu.VMEM((CH, DIM), jnp.float32),
        pltpu.VMEM((CH, DIM), jnp.float32),
        pltpu.SemaphoreType.DMA,
        pltpu.SemaphoreType.DMA,
        pltpu.SemaphoreType.DMA,
        pltpu.SemaphoreType.DMA,
    ],
)
def _gather(idx_hbm, table_hbm, out_hbm, idx_v, buf0, buf1, g0, g1, s0, s1):
    wid = lax.axis_index("s") * NC + lax.axis_index("c")
    base = wid * BPW
    pltpu.sync_copy(idx_hbm.at[pl.ds(base, BPW)], idx_v)

    def gather_start(c, buf, sem):
        pltpu.make_async_copy(
            table_hbm.at[idx_v.at[pl.ds(c * CH, CH)]], buf, sem
        ).start()

    def gather_wait(c, buf, sem):
        pltpu.make_async_copy(
            table_hbm.at[idx_v.at[pl.ds(c * CH, CH)]], buf, sem
        ).wait()

    def scatter_start(c, buf, sem):
        pltpu.make_async_copy(
            buf, out_hbm.at[pl.ds(base + c * CH, CH)], sem
        ).start()

    def scatter_wait(c, buf, sem):
        pltpu.make_async_copy(
            buf, out_hbm.at[pl.ds(base + c * CH, CH)], sem
        ).wait()

    gather_start(0, buf0, g0)
    gather_start(1, buf1, g1)

    def body(g, carry):
        c0 = 2 * g
        c1 = c0 + 1
        gather_wait(c0, buf0, g0)
        scatter_start(c0, buf0, s0)
        gather_wait(c1, buf1, g1)
        scatter_start(c1, buf1, s1)
        scatter_wait(c0, buf0, s0)
        gather_start(c0 + 2, buf0, g0)
        scatter_wait(c1, buf1, s1)
        gather_start(c1 + 2, buf1, g1)
        return carry

    lax.fori_loop(0, NCHUNK // 2 - 1, body, 0)

    c0 = NCHUNK - 2
    c1 = NCHUNK - 1
    gather_wait(c0, buf0, g0)
    scatter_start(c0, buf0, s0)
    gather_wait(c1, buf1, g1)
    scatter_start(c1, buf1, s1)
    scatter_wait(c0, buf0, s0)
    scatter_wait(c1, buf1, s1)


def kernel(path, table):
    flat = path.reshape(-1).astype(jnp.int32)
    out = _gather(flat, table)
    out4 = out.reshape(path.shape[0], path.shape[1], path.shape[2], DIM)
    return tuple(out4[i] for i in range(path.shape[0]))


# speedup vs baseline: 1.5818x; 1.4309x over previous
import functools

import jax
import jax.numpy as jnp
from jax import lax
from jax.experimental import pallas as pl
from jax.experimental.pallas import tpu as pltpu
from jax.experimental.pallas import tpu_sc as plsc

VOCAB = 361
DIM = 512
NC, NS = 2, 16
NW = NC * NS
NSEG = 4
SEG = 1024 * 50
WPS = NW // NSEG
BPW = SEG // WPS
CH = 80
NCHUNK = BPW // CH

_mesh = plsc.VectorSubcoreMesh(core_axis_name="c", subcore_axis_name="s")


@functools.partial(
    pl.kernel,
    mesh=_mesh,
    out_type=[jax.ShapeDtypeStruct((SEG, DIM), jnp.float32) for _ in range(NSEG)],
    scratch_types=[
        pltpu.VMEM((BPW,), jnp.int32),
        pltpu.VMEM((CH, DIM), jnp.float32),
        pltpu.VMEM((CH, DIM), jnp.float32),
        pltpu.SemaphoreType.DMA,
        pltpu.SemaphoreType.DMA,
        pltpu.SemaphoreType.DMA,
        pltpu.SemaphoreType.DMA,
    ],
)
def _gather(idx_hbm, table_hbm, o0, o1, o2, o3, idx_v, buf0, buf1, g0, g1, s0, s1):
    wid = lax.axis_index("s") * NC + lax.axis_index("c")
    pltpu.sync_copy(idx_hbm.at[pl.ds(wid * BPW, BPW)], idx_v)

    def gather_start(c, buf, sem):
        pltpu.make_async_copy(
            table_hbm.at[idx_v.at[pl.ds(c * CH, CH)]], buf, sem
        ).start()

    def gather_wait(c, buf, sem):
        pltpu.make_async_copy(
            table_hbm.at[idx_v.at[pl.ds(c * CH, CH)]], buf, sem
        ).wait()

    for seg, out_hbm in enumerate((o0, o1, o2, o3)):

        @pl.when(wid // WPS == seg)
        def _():
            base = (wid - seg * WPS) * BPW

            def scatter_start(c, buf, sem):
                pltpu.make_async_copy(
                    buf, out_hbm.at[pl.ds(base + c * CH, CH)], sem
                ).start()

            def scatter_wait(c, buf, sem):
                pltpu.make_async_copy(
                    buf, out_hbm.at[pl.ds(base + c * CH, CH)], sem
                ).wait()

            gather_start(0, buf0, g0)
            gather_start(1, buf1, g1)

            def body(g, carry):
                c0 = 2 * g
                c1 = c0 + 1
                gather_wait(c0, buf0, g0)
                scatter_start(c0, buf0, s0)
                gather_wait(c1, buf1, g1)
                scatter_start(c1, buf1, s1)
                scatter_wait(c0, buf0, s0)
                gather_start(c0 + 2, buf0, g0)
                scatter_wait(c1, buf1, s1)
                gather_start(c1 + 2, buf1, g1)
                return carry

            lax.fori_loop(0, NCHUNK // 2 - 1, body, 0)

            c0 = NCHUNK - 2
            c1 = NCHUNK - 1
            gather_wait(c0, buf0, g0)
            scatter_start(c0, buf0, s0)
            gather_wait(c1, buf1, g1)
            scatter_start(c1, buf1, s1)
            scatter_wait(c0, buf0, s0)
            scatter_wait(c1, buf1, s1)


def kernel(path, table):
    idx = path.reshape(-1).astype(jnp.int32)
    outs = _gather(idx, table)
    return tuple(o.reshape(path.shape[1], path.shape[2], DIM) for o in outs)


# speedup vs baseline: 2.0692x; 1.3081x over previous
import functools

import jax
import jax.numpy as jnp
from jax import lax
from jax.experimental import pallas as pl
from jax.experimental.pallas import tpu as pltpu
from jax.experimental.pallas import tpu_sc as plsc

VOCAB = 361
DIM = 512
NC, NS = 2, 16
NW = NC * NS
NSEG = 4
NPATH = 1024
PLEN = 50
PMAIN = 48
PTAIL = PLEN - PMAIN
PPAD = 64
WPS = NW // NSEG
PPW = NPATH // WPS
GRP = 8 // PTAIL
NGRP = PPW // GRP

_mesh = plsc.VectorSubcoreMesh(core_axis_name="c", subcore_axis_name="s")


@functools.partial(
    pl.kernel,
    mesh=_mesh,
    out_type=(
        [jax.ShapeDtypeStruct((NPATH, PLEN, DIM), jnp.float32) for _ in range(NSEG)]
        + [jax.ShapeDtypeStruct((NSEG * NPATH * PTAIL, DIM), jnp.float32)]
    ),
    scratch_types=[
        pltpu.VMEM((PPW, PPAD), jnp.int32),
        pltpu.VMEM((NGRP * 8,), jnp.int32),
        pltpu.VMEM((PLEN, DIM), jnp.float32),
        pltpu.VMEM((PLEN, DIM), jnp.float32),
        pltpu.VMEM((8, DIM), jnp.float32),
        pltpu.VMEM((8, DIM), jnp.float32),
        pltpu.SemaphoreType.DMA,
        pltpu.SemaphoreType.DMA,
        pltpu.SemaphoreType.DMA,
        pltpu.SemaphoreType.DMA,
    ],
)
def _gather(idx_hbm, tidx_hbm, table_hbm, o0, o1, o2, o3, tails,
            idx_v, tidx_v, buf0, buf1, tb0, tb1, g0, g1, s0, s1):
    wid = lax.axis_index("s") * NC + lax.axis_index("c")
    pltpu.sync_copy(idx_hbm.at[pl.ds(wid * PPW, PPW)], idx_v)
    pltpu.sync_copy(tidx_hbm.at[pl.ds(wid * NGRP * 8, NGRP * 8)], tidx_v)

    def gather_start(p, buf, sem):
        pltpu.make_async_copy(
            table_hbm.at[idx_v.at[p].at[pl.ds(0, PLEN)]], buf, sem
        ).start()

    def gather_wait(p, buf, sem):
        pltpu.make_async_copy(
            table_hbm.at[idx_v.at[p].at[pl.ds(0, PLEN)]], buf, sem
        ).wait()

    for seg, out_hbm in enumerate((o0, o1, o2, o3)):

        @pl.when(wid // WPS == seg)
        def _():
            base = (wid - seg * WPS) * PPW

            def scatter_start(p, buf, sem):
                pltpu.make_async_copy(
                    buf.at[pl.ds(0, PMAIN)],
                    out_hbm.at[base + p].at[pl.ds(0, PMAIN)],
                    sem,
                ).start()

            def scatter_wait(p, buf, sem):
                pltpu.make_async_copy(
                    buf.at[pl.ds(0, PMAIN)],
                    out_hbm.at[base + p].at[pl.ds(0, PMAIN)],
                    sem,
                ).wait()

            gather_start(0, buf0, g0)
            gather_start(1, buf1, g1)

            def body(g, carry):
                p0 = 2 * g
                p1 = p0 + 1
                gather_wait(p0, buf0, g0)
                scatter_start(p0, buf0, s0)
                gather_wait(p1, buf1, g1)
                scatter_start(p1, buf1, s1)
                scatter_wait(p0, buf0, s0)
                gather_start(p0 + 2, buf0, g0)
                scatter_wait(p1, buf1, s1)
                gather_start(p1 + 2, buf1, g1)
                return carry

            lax.fori_loop(0, PPW // 2 - 1, body, 0)

            p0 = PPW - 2
            p1 = PPW - 1
            gather_wait(p0, buf0, g0)
            scatter_start(p0, buf0, s0)
            gather_wait(p1, buf1, g1)
            scatter_start(p1, buf1, s1)
            scatter_wait(p0, buf0, s0)
            scatter_wait(p1, buf1, s1)

    tbase = wid * NGRP * 8

    def tg_start(k, buf, sem):
        pltpu.make_async_copy(
            table_hbm.at[tidx_v.at[pl.ds(k * 8, 8)]], buf, sem
        ).start()

    def tg_wait(k, buf, sem):
        pltpu.make_async_copy(
            table_hbm.at[tidx_v.at[pl.ds(k * 8, 8)]], buf, sem
        ).wait()

    def ts_start(k, buf, sem):
        pltpu.make_async_copy(buf, tails.at[pl.ds(tbase + k * 8, 8)], sem).start()

    def ts_wait(k, buf, sem):
        pltpu.make_async_copy(buf, tails.at[pl.ds(tbase + k * 8, 8)], sem).wait()

    tg_start(0, tb0, g0)
    tg_start(1, tb1, g1)

    def tbody(g, carry):
        k0 = 2 * g
        k1 = k0 + 1
        tg_wait(k0, tb0, g0)
        ts_start(k0, tb0, s0)
        tg_wait(k1, tb1, g1)
        ts_start(k1, tb1, s1)
        ts_wait(k0, tb0, s0)
        tg_start(k0 + 2, tb0, g0)
        ts_wait(k1, tb1, s1)
        tg_start(k1 + 2, tb1, g1)
        return carry

    lax.fori_loop(0, NGRP // 2 - 1, tbody, 0)
    k0 = NGRP - 2
    k1 = NGRP - 1
    tg_wait(k0, tb0, g0)
    ts_start(k0, tb0, s0)
    tg_wait(k1, tb1, g1)
    ts_start(k1, tb1, s1)
    ts_wait(k0, tb0, s0)
    ts_wait(k1, tb1, s1)


def kernel(path, table):
    idx = path.reshape(NSEG * NPATH, PLEN).astype(jnp.int32)
    idx_pad = jnp.pad(idx, ((0, 0), (0, PPAD - PLEN)))
    tidx = idx[:, PMAIN:].reshape(-1)
    *outs, tails = _gather(idx_pad, tidx, table)
    tails = tails.reshape(NSEG, NPATH, PTAIL, DIM)
    return tuple(
        lax.dynamic_update_slice(o, tails[i], (0, PMAIN, 0))
        for i, o in enumerate(outs)
    )
